# baseline (speedup 1.0000x reference)
import functools

import jax
import jax.numpy as jnp
from jax import lax
from jax.experimental import pallas as pl
from jax.experimental.pallas import tpu as pltpu
from jax.experimental.pallas import tpu_sc as plsc


def _qk_body(x_ref, wv_ref, wqt_ref, wkt_ref, bv_ref, bq_ref, bk_ref,
             out_ref):
    d = wqt_ref.shape[1]
    wqk = jnp.transpose(jnp.concatenate(
        [wqt_ref[...], wkt_ref[...], jnp.zeros((6, d), jnp.float32)],
        axis=0))
    w2 = jnp.dot(wv_ref[...], wqk, preferred_element_type=jnp.float32)
    bqk = jnp.concatenate(
        [bq_ref[...], bk_ref[...], jnp.zeros((6, 1), jnp.float32)], axis=0)
    b2 = lax.dot_general(
        wqk, bv_ref[...], (((0,), (1,)), ((), ())),
        preferred_element_type=jnp.float32) + bqk
    n = x_ref.shape[0]
    out_ref[:, :n] = lax.dot_general(
        w2, x_ref[...], (((0,), (1,)), ((), ())),
        preferred_element_type=jnp.float32) + b2


def _make_edge_kernel(n_tiles, n_edges, n_chunks, chunks_per_worker):
    mesh = plsc.VectorSubcoreMesh(
        core_axis_name="c", subcore_axis_name="s", num_cores=1)
    num_cores = 1
    epw = chunks_per_worker * 128

    @functools.partial(
        pl.kernel,
        mesh=mesh,
        out_type=jax.ShapeDtypeStruct((1, n_edges), jnp.float32),
        compiler_params=pltpu.CompilerParams(
            needs_layout_passes=False,
            use_tc_tiling_on_sc=False,
        ),
        scratch_types=[
            pltpu.VMEM((n_tiles, 2, 128), jnp.float32),
            pltpu.VMEM((chunks_per_worker, 2, 128), jnp.int32),
            pltpu.VMEM((epw,), jnp.float32),
            pltpu.SemaphoreType.DMA,
        ],
    )
    def edge_kernel(qk_hbm, ei_hbm, out_hbm, qk_v, ei_v, out_v, sem):
        wid = lax.axis_index("s") * num_cores + lax.axis_index("c")
        base_c = jnp.minimum(
            wid * chunks_per_worker, n_chunks - chunks_per_worker)
        cp_qk = pltpu.async_copy(qk_hbm.at[:, 0:2, :], qk_v, sem)
        cp_e = pltpu.async_copy(
            ei_hbm.at[pl.ds(base_c, chunks_per_worker)], ei_v, sem)
        cp_qk.wait()
        cp_e.wait()
        zero16 = jnp.zeros((16,), jnp.int32)
        one16 = jnp.ones((16,), jnp.int32)

        @plsc.parallel_loop(0, epw, step=16, unroll=2)
        def _(off):
            c = off >> 7
            p = off & 127
            s_idx = ei_v[c, 0, pl.ds(p, 16)]
            d_idx = ei_v[c, 1, pl.ds(p, 16)]
            e = (plsc.load_gather(qk_v, [s_idx >> 7, zero16, s_idx & 127])
                 + plsc.load_gather(qk_v, [d_idx >> 7, one16, d_idx & 127]))
            out_v[pl.ds(off, 16)] = jnp.where(e > 0, e, 0.2 * e)

        pltpu.sync_copy(
            out_v, out_hbm.at[0, pl.ds(base_c * 128, epw)])

    return edge_kernel


def kernel(x, edge_index, Wv, bv, Wq, bq, Wk, bk):
    n, d_in = x.shape
    e = edge_index.shape[1]

    n_tiles = (n + 127) // 128
    np_ = n_tiles * 128
    qk = pl.pallas_call(
        _qk_body,
        out_shape=jax.ShapeDtypeStruct((8, np_), jnp.float32),
    )(x, Wv, Wq.reshape(1, -1), Wk.reshape(1, -1),
      bv.reshape(1, d_in), bq.reshape(1, 1), bk.reshape(1, 1))
    qk3 = qk.reshape(8, n_tiles, 128).transpose(1, 0, 2)

    ei = edge_index.astype(jnp.int32)
    n_chunks = e // 128
    ei3 = ei.reshape(2, n_chunks, 128).transpose(1, 0, 2)

    n_workers = 16
    cpw = -(-n_chunks // n_workers)
    cpw += cpw % 2
    edge_kernel = _make_edge_kernel(n_tiles, e, n_chunks, cpw)
    return edge_kernel(qk3, ei3).T


# speedup vs baseline: 1.0219x; 1.0219x over previous
import functools

import jax
import jax.numpy as jnp
from jax import lax
from jax.experimental import pallas as pl
from jax.experimental.pallas import tpu as pltpu
from jax.experimental.pallas import tpu_sc as plsc


def _qk_body(x_ref, wv_ref, wqt_ref, wkt_ref, bv_ref, bq_ref, bk_ref,
             out_ref):
    d = wqt_ref.shape[1]
    wqk = jnp.transpose(jnp.concatenate(
        [wqt_ref[...], wkt_ref[...], jnp.zeros((6, d), jnp.float32)],
        axis=0))
    w2 = jnp.dot(wv_ref[...], wqk, preferred_element_type=jnp.float32)
    bqk = jnp.concatenate(
        [bq_ref[...], bk_ref[...], jnp.zeros((6, 1), jnp.float32)], axis=0)
    b2 = lax.dot_general(
        wqk, bv_ref[...], (((0,), (1,)), ((), ())),
        preferred_element_type=jnp.float32) + bqk
    n = x_ref.shape[0]
    out_ref[:, :n] = lax.dot_general(
        w2, x_ref[...], (((0,), (1,)), ((), ())),
        preferred_element_type=jnp.float32) + b2


def _make_edge_kernel(n_tiles, n_edges, n_chunks, chunks_per_worker):
    mesh = plsc.VectorSubcoreMesh(
        core_axis_name="c", subcore_axis_name="s", num_cores=1)
    num_cores = 1
    epw = chunks_per_worker * 128

    @functools.partial(
        pl.kernel,
        mesh=mesh,
        out_type=jax.ShapeDtypeStruct((1, n_edges), jnp.float32),
        compiler_params=pltpu.CompilerParams(
            needs_layout_passes=False,
            use_tc_tiling_on_sc=False,
        ),
        scratch_types=[
            pltpu.VMEM((n_tiles, 2, 128), jnp.float32),
            pltpu.VMEM((chunks_per_worker, 2, 128), jnp.int32),
            pltpu.VMEM((epw,), jnp.float32),
            pltpu.SemaphoreType.DMA,
            pltpu.SemaphoreType.DMA,
        ],
    )
    def edge_kernel(qk_hbm, ei_hbm, out_hbm, qk_v, ei_v, out_v, sem, sem2):
        wid = lax.axis_index("s") * num_cores + lax.axis_index("c")
        base_c = jnp.minimum(
            wid * chunks_per_worker, n_chunks - chunks_per_worker)
        half_c = chunks_per_worker // 2
        half_e = half_c * 128
        cp_qk = pltpu.async_copy(qk_hbm.at[:, 0:2, :], qk_v, sem)
        cp_e0 = pltpu.async_copy(
            ei_hbm.at[pl.ds(base_c, half_c)], ei_v.at[:half_c], sem)
        cp_e1 = pltpu.async_copy(
            ei_hbm.at[pl.ds(base_c + half_c, half_c)],
            ei_v.at[half_c:], sem2)
        cp_qk.wait()
        cp_e0.wait()
        zero16 = jnp.zeros((16,), jnp.int32)
        one16 = jnp.ones((16,), jnp.int32)

        @plsc.parallel_loop(0, half_e, step=16, unroll=4)
        def _(off):
            c = off >> 7
            p = off & 127
            s_idx = ei_v[c, 0, pl.ds(p, 16)]
            d_idx = ei_v[c, 1, pl.ds(p, 16)]
            e = (plsc.load_gather(qk_v, [s_idx >> 7, zero16, s_idx & 127])
                 + plsc.load_gather(qk_v, [d_idx >> 7, one16, d_idx & 127]))
            out_v[pl.ds(off, 16)] = jnp.where(e > 0, e, 0.2 * e)

        cp_o0 = pltpu.async_copy(
            out_v.at[:half_e], out_hbm.at[0, pl.ds(base_c * 128, half_e)],
            sem)
        cp_e1.wait()

        @plsc.parallel_loop(half_e, epw, step=16, unroll=4)
        def _(off):
            c = off >> 7
            p = off & 127
            s_idx = ei_v[c, 0, pl.ds(p, 16)]
            d_idx = ei_v[c, 1, pl.ds(p, 16)]
            e = (plsc.load_gather(qk_v, [s_idx >> 7, zero16, s_idx & 127])
                 + plsc.load_gather(qk_v, [d_idx >> 7, one16, d_idx & 127]))
            out_v[pl.ds(off, 16)] = jnp.where(e > 0, e, 0.2 * e)

        cp_o0.wait()
        pltpu.sync_copy(
            out_v.at[half_e:],
            out_hbm.at[0, pl.ds(base_c * 128 + half_e, half_e)])

    return edge_kernel


def kernel(x, edge_index, Wv, bv, Wq, bq, Wk, bk):
    n, d_in = x.shape
    e = edge_index.shape[1]

    n_tiles = (n + 127) // 128
    np_ = n_tiles * 128
    qk = pl.pallas_call(
        _qk_body,
        out_shape=jax.ShapeDtypeStruct((8, np_), jnp.float32),
    )(x, Wv, Wq.reshape(1, -1), Wk.reshape(1, -1),
      bv.reshape(1, d_in), bq.reshape(1, 1), bk.reshape(1, 1))
    qk3 = qk.reshape(8, n_tiles, 128).transpose(1, 0, 2)

    ei = edge_index.astype(jnp.int32)
    n_chunks = e // 128
    ei3 = ei.reshape(2, n_chunks, 128).transpose(1, 0, 2)

    n_workers = 16
    cpw = -(-n_chunks // n_workers)
    cpw += cpw % 2
    edge_kernel = _make_edge_kernel(n_tiles, e, n_chunks, cpw)
    return edge_kernel(qk3, ei3).T


# speedup vs baseline: 1.0310x; 1.0089x over previous
import functools

import jax
import jax.numpy as jnp
from jax import lax
from jax.experimental import pallas as pl
from jax.experimental.pallas import tpu as pltpu
from jax.experimental.pallas import tpu_sc as plsc


def _qk_body(x_ref, wv_ref, wqt_ref, wkt_ref, bv_ref, bq_ref, bk_ref,
             out_ref):
    d = wqt_ref.shape[1]
    wqk = jnp.transpose(jnp.concatenate(
        [wqt_ref[...], wkt_ref[...], jnp.zeros((6, d), jnp.float32)],
        axis=0))
    w2 = jnp.dot(wv_ref[...], wqk, preferred_element_type=jnp.float32)
    bqk = jnp.concatenate(
        [bq_ref[...], bk_ref[...], jnp.zeros((6, 1), jnp.float32)], axis=0)
    b2 = lax.dot_general(
        wqk, bv_ref[...], (((0,), (1,)), ((), ())),
        preferred_element_type=jnp.float32) + bqk
    n = x_ref.shape[0]
    out_ref[:, :n] = lax.dot_general(
        w2, x_ref[...], (((0,), (1,)), ((), ())),
        preferred_element_type=jnp.float32) + b2


def _make_edge_kernel(n_tiles, n_edges, n_chunks, chunks_per_worker):
    mesh = plsc.VectorSubcoreMesh(
        core_axis_name="c", subcore_axis_name="s", num_cores=1)
    num_cores = 1
    epw = chunks_per_worker * 128

    @functools.partial(
        pl.kernel,
        mesh=mesh,
        out_type=jax.ShapeDtypeStruct((1, n_edges), jnp.float32),
        compiler_params=pltpu.CompilerParams(
            needs_layout_passes=False,
            use_tc_tiling_on_sc=False,
        ),
        scratch_types=[
            pltpu.VMEM((n_tiles, 2, 128), jnp.float32),
            pltpu.VMEM((chunks_per_worker, 2, 128), jnp.int32),
            pltpu.VMEM((epw,), jnp.float32),
            pltpu.SemaphoreType.DMA,
            pltpu.SemaphoreType.DMA,
        ],
    )
    def edge_kernel(qk_hbm, ei_hbm, out_hbm, qk_v, ei_v, out_v, sem, sem2):
        wid = lax.axis_index("s") * num_cores + lax.axis_index("c")
        base_c = jnp.minimum(
            wid * chunks_per_worker, n_chunks - chunks_per_worker)
        half_c = chunks_per_worker // 2
        half_e = half_c * 128
        cp_qk = pltpu.async_copy(qk_hbm.at[:, 0:2, :], qk_v, sem)
        cp_e0 = pltpu.async_copy(
            ei_hbm.at[pl.ds(base_c, half_c)], ei_v.at[:half_c], sem)
        cp_e1 = pltpu.async_copy(
            ei_hbm.at[pl.ds(base_c + half_c, half_c)],
            ei_v.at[half_c:], sem2)
        cp_qk.wait()
        cp_e0.wait()
        zero16 = jnp.zeros((16,), jnp.int32)
        one16 = jnp.ones((16,), jnp.int32)

        @plsc.parallel_loop(0, half_e, step=16, unroll=8)
        def _(off):
            c = off >> 7
            p = off & 127
            s_idx = ei_v[c, 0, pl.ds(p, 16)]
            d_idx = ei_v[c, 1, pl.ds(p, 16)]
            e = (plsc.load_gather(qk_v, [s_idx >> 7, zero16, s_idx & 127])
                 + plsc.load_gather(qk_v, [d_idx >> 7, one16, d_idx & 127]))
            out_v[pl.ds(off, 16)] = jnp.where(e > 0, e, 0.2 * e)

        cp_o0 = pltpu.async_copy(
            out_v.at[:half_e], out_hbm.at[0, pl.ds(base_c * 128, half_e)],
            sem)
        cp_e1.wait()

        @plsc.parallel_loop(half_e, epw, step=16, unroll=8)
        def _(off):
            c = off >> 7
            p = off & 127
            s_idx = ei_v[c, 0, pl.ds(p, 16)]
            d_idx = ei_v[c, 1, pl.ds(p, 16)]
            e = (plsc.load_gather(qk_v, [s_idx >> 7, zero16, s_idx & 127])
                 + plsc.load_gather(qk_v, [d_idx >> 7, one16, d_idx & 127]))
            out_v[pl.ds(off, 16)] = jnp.where(e > 0, e, 0.2 * e)

        cp_o0.wait()
        pltpu.sync_copy(
            out_v.at[half_e:],
            out_hbm.at[0, pl.ds(base_c * 128 + half_e, half_e)])

    return edge_kernel


def kernel(x, edge_index, Wv, bv, Wq, bq, Wk, bk):
    n, d_in = x.shape
    e = edge_index.shape[1]

    n_tiles = (n + 127) // 128
    np_ = n_tiles * 128
    qk = pl.pallas_call(
        _qk_body,
        out_shape=jax.ShapeDtypeStruct((8, np_), jnp.float32),
    )(x, Wv, Wq.reshape(1, -1), Wk.reshape(1, -1),
      bv.reshape(1, d_in), bq.reshape(1, 1), bk.reshape(1, 1))
    qk3 = qk.reshape(8, n_tiles, 128).transpose(1, 0, 2)

    ei = edge_index.astype(jnp.int32)
    n_chunks = e // 128
    ei3 = ei.reshape(2, n_chunks, 128).transpose(1, 0, 2)

    n_workers = 16
    cpw = -(-n_chunks // n_workers)
    cpw += cpw % 2
    edge_kernel = _make_edge_kernel(n_tiles, e, n_chunks, cpw)
    return edge_kernel(qk3, ei3).T
